# baseline (speedup 1.0000x reference)
import jax
import jax.numpy as jnp
from jax import lax
from jax.experimental import pallas as pl
from jax.experimental.pallas import tpu as pltpu

BATCH = 2048
N_ENT = 1024
D_DIM = 32
P_SENT = 1024
ROWS = 16
_EPS = 1e-12


def _body(starts_ref, counts_ref, perm_ref,
          e_ref, ut_ref, vt_ref, w_ref, h_ref, k_ref, out_ref, ew_scr):
    i = pl.program_id(0)

    @pl.when(i == 0)
    def _():
        ew_scr[...] = jnp.dot(e_ref[...], w_ref[...],
                              preferred_element_type=jnp.float32)

    for r in range(ROWS):
        b = i * ROWS + r
        cnt = counts_ref[b]
        s0 = starts_ref[b]
        hT = h_ref[r]

        def hit_fn(hT=hT, r=r, cnt=cnt, s0=s0):
            kT = k_ref[r]
            baseT = (jnp.dot(ut_ref[...], hT,
                             preferred_element_type=jnp.float32)
                     + jnp.dot(vt_ref[...], kT,
                               preferred_element_type=jnp.float32))
            sT = hT + kT

            def loop(j, acc):
                p = perm_ref[j]
                e_row = e_ref[pl.ds(p, 1), :]
                ewT = lax.transpose(ew_scr[pl.ds(p, 1), :], (1, 0))
                logits = jnp.dot(e_row, sT,
                                 preferred_element_type=jnp.float32)
                gate = jax.nn.sigmoid(logits)
                htld = jnp.maximum(baseT + ewT, 0.0)
                return acc + gate * htld

            acc = lax.fori_loop(s0, s0 + cnt, loop,
                                jnp.zeros((D_DIM, N_ENT), jnp.float32))
            return hT + acc

        xT = lax.cond(cnt > 0, hit_fn, lambda hT=hT: hT)
        ss = jnp.dot(jnp.ones((1, D_DIM), jnp.float32), xT * xT,
                     preferred_element_type=jnp.float32)
        out_ref[r] = xT * lax.rsqrt(jnp.maximum(ss, _EPS))


def kernel(encoded_sents, indices, hiddens, keys, U, V, W):
    ht = jnp.transpose(hiddens, (0, 2, 1))
    kt = jnp.transpose(keys, (0, 2, 1))

    perm = jnp.argsort(indices).astype(jnp.int32)
    sidx = jnp.take(indices, perm)
    sp = jnp.searchsorted(sidx, jnp.arange(BATCH + 1, dtype=jnp.int32),
                          side="left").astype(jnp.int32)
    starts = sp[:BATCH]
    counts = sp[1:] - starts

    grid_spec = pltpu.PrefetchScalarGridSpec(
        num_scalar_prefetch=3,
        grid=(BATCH // ROWS,),
        in_specs=[
            pl.BlockSpec((P_SENT, D_DIM), lambda i, *_: (0, 0)),
            pl.BlockSpec((D_DIM, D_DIM), lambda i, *_: (0, 0)),
            pl.BlockSpec((D_DIM, D_DIM), lambda i, *_: (0, 0)),
            pl.BlockSpec((D_DIM, D_DIM), lambda i, *_: (0, 0)),
            pl.BlockSpec((ROWS, D_DIM, N_ENT), lambda i, *_: (i, 0, 0)),
            pl.BlockSpec((ROWS, D_DIM, N_ENT), lambda i, *_: (i, 0, 0)),
        ],
        out_specs=pl.BlockSpec((ROWS, D_DIM, N_ENT), lambda i, *_: (i, 0, 0)),
        scratch_shapes=[pltpu.VMEM((P_SENT, D_DIM), jnp.float32)],
    )
    outT = pl.pallas_call(
        _body,
        grid_spec=grid_spec,
        out_shape=jax.ShapeDtypeStruct((BATCH, D_DIM, N_ENT), jnp.float32),
        compiler_params=pltpu.CompilerParams(
            dimension_semantics=("arbitrary",)),
    )(starts, counts, perm, encoded_sents, U.T, V.T, W, ht, kt)

    return jnp.transpose(outT, (0, 2, 1))


# speedup vs baseline: 1.3789x; 1.3789x over previous
import jax
import jax.numpy as jnp
from jax import lax
from jax.experimental import pallas as pl
from jax.experimental.pallas import tpu as pltpu

BATCH = 2048
N_ENT = 1024
D_DIM = 32
P_SENT = 1024
ROWS = 32
_EPS = 1e-12


def _body(starts_ref, counts_ref, perm_ref,
          e_ref, ut_ref, vt_ref, w_ref, h_ref, k_ref, out_ref, ew_scr):
    i = pl.program_id(0)

    @pl.when(i == 0)
    def _():
        ew_scr[...] = jnp.dot(e_ref[...], w_ref[...],
                              preferred_element_type=jnp.float32)

    for r in range(ROWS):
        b = i * ROWS + r
        cnt = counts_ref[b]
        s0 = starts_ref[b]
        hT = h_ref[r]

        def hit_fn(hT=hT, r=r, cnt=cnt, s0=s0):
            kT = k_ref[r]
            baseT = (jnp.dot(ut_ref[...], hT,
                             preferred_element_type=jnp.float32)
                     + jnp.dot(vt_ref[...], kT,
                               preferred_element_type=jnp.float32))
            sT = hT + kT

            def loop(j, acc):
                p = perm_ref[j]
                e_row = e_ref[pl.ds(p, 1), :]
                ewT = lax.transpose(ew_scr[pl.ds(p, 1), :], (1, 0))
                logits = jnp.dot(e_row, sT,
                                 preferred_element_type=jnp.float32)
                gate = jax.nn.sigmoid(logits)
                htld = jnp.maximum(baseT + ewT, 0.0)
                return acc + gate * htld

            acc = lax.fori_loop(s0, s0 + cnt, loop,
                                jnp.zeros((D_DIM, N_ENT), jnp.float32))
            return hT + acc

        xT = lax.cond(cnt > 0, hit_fn, lambda hT=hT: hT)
        ss = jnp.sum(xT * xT, axis=0, keepdims=True)
        out_ref[r] = xT * lax.rsqrt(jnp.maximum(ss, _EPS))


def kernel(encoded_sents, indices, hiddens, keys, U, V, W):
    ht = jnp.transpose(hiddens, (0, 2, 1))
    kt = jnp.transpose(keys, (0, 2, 1))

    perm = jnp.argsort(indices).astype(jnp.int32)
    sidx = jnp.take(indices, perm)
    sp = jnp.searchsorted(sidx, jnp.arange(BATCH + 1, dtype=jnp.int32),
                          side="left").astype(jnp.int32)
    starts = sp[:BATCH]
    counts = sp[1:] - starts

    grid_spec = pltpu.PrefetchScalarGridSpec(
        num_scalar_prefetch=3,
        grid=(BATCH // ROWS,),
        in_specs=[
            pl.BlockSpec((P_SENT, D_DIM), lambda i, *_: (0, 0)),
            pl.BlockSpec((D_DIM, D_DIM), lambda i, *_: (0, 0)),
            pl.BlockSpec((D_DIM, D_DIM), lambda i, *_: (0, 0)),
            pl.BlockSpec((D_DIM, D_DIM), lambda i, *_: (0, 0)),
            pl.BlockSpec((ROWS, D_DIM, N_ENT), lambda i, *_: (i, 0, 0)),
            pl.BlockSpec((ROWS, D_DIM, N_ENT), lambda i, *_: (i, 0, 0)),
        ],
        out_specs=pl.BlockSpec((ROWS, D_DIM, N_ENT), lambda i, *_: (i, 0, 0)),
        scratch_shapes=[pltpu.VMEM((P_SENT, D_DIM), jnp.float32)],
    )
    outT = pl.pallas_call(
        _body,
        grid_spec=grid_spec,
        out_shape=jax.ShapeDtypeStruct((BATCH, D_DIM, N_ENT), jnp.float32),
        compiler_params=pltpu.CompilerParams(
            dimension_semantics=("arbitrary",)),
    )(starts, counts, perm, encoded_sents, U.T, V.T, W, ht, kt)

    return jnp.transpose(outT, (0, 2, 1))


# speedup vs baseline: 1.3940x; 1.0109x over previous
import jax
import jax.numpy as jnp
from jax import lax
from jax.experimental import pallas as pl
from jax.experimental.pallas import tpu as pltpu

BATCH = 2048
N_ENT = 1024
D_DIM = 32
P_SENT = 1024
ROWS = 64
_EPS = 1e-12


def _body(starts_ref, counts_ref, perm_ref,
          e_ref, ut_ref, vt_ref, w_ref, h_ref, k_ref, out_ref, ew_scr):
    i = pl.program_id(0)

    @pl.when(i == 0)
    def _():
        ew_scr[...] = jnp.dot(e_ref[...], w_ref[...],
                              preferred_element_type=jnp.float32)

    for r in range(ROWS):
        b = i * ROWS + r
        cnt = counts_ref[b]
        s0 = starts_ref[b]
        hT = h_ref[r]

        def hit_fn(hT=hT, r=r, cnt=cnt, s0=s0):
            kT = k_ref[r]
            baseT = (jnp.dot(ut_ref[...], hT,
                             preferred_element_type=jnp.float32)
                     + jnp.dot(vt_ref[...], kT,
                               preferred_element_type=jnp.float32))
            sT = hT + kT

            def loop(j, acc):
                p = perm_ref[j]
                e_row = e_ref[pl.ds(p, 1), :]
                ewT = lax.transpose(ew_scr[pl.ds(p, 1), :], (1, 0))
                logits = jnp.dot(e_row, sT,
                                 preferred_element_type=jnp.float32)
                gate = jax.nn.sigmoid(logits)
                htld = jnp.maximum(baseT + ewT, 0.0)
                return acc + gate * htld

            acc = lax.fori_loop(s0, s0 + cnt, loop,
                                jnp.zeros((D_DIM, N_ENT), jnp.float32))
            return hT + acc

        xT = lax.cond(cnt > 0, hit_fn, lambda hT=hT: hT)
        ss = jnp.sum(xT * xT, axis=0, keepdims=True)
        out_ref[r] = xT * lax.rsqrt(jnp.maximum(ss, _EPS))


def kernel(encoded_sents, indices, hiddens, keys, U, V, W):
    ht = jnp.transpose(hiddens, (0, 2, 1))
    kt = jnp.transpose(keys, (0, 2, 1))

    perm = jnp.argsort(indices).astype(jnp.int32)
    sidx = jnp.take(indices, perm)
    sp = jnp.searchsorted(sidx, jnp.arange(BATCH + 1, dtype=jnp.int32),
                          side="left").astype(jnp.int32)
    starts = sp[:BATCH]
    counts = sp[1:] - starts

    grid_spec = pltpu.PrefetchScalarGridSpec(
        num_scalar_prefetch=3,
        grid=(BATCH // ROWS,),
        in_specs=[
            pl.BlockSpec((P_SENT, D_DIM), lambda i, *_: (0, 0)),
            pl.BlockSpec((D_DIM, D_DIM), lambda i, *_: (0, 0)),
            pl.BlockSpec((D_DIM, D_DIM), lambda i, *_: (0, 0)),
            pl.BlockSpec((D_DIM, D_DIM), lambda i, *_: (0, 0)),
            pl.BlockSpec((ROWS, D_DIM, N_ENT), lambda i, *_: (i, 0, 0)),
            pl.BlockSpec((ROWS, D_DIM, N_ENT), lambda i, *_: (i, 0, 0)),
        ],
        out_specs=pl.BlockSpec((ROWS, D_DIM, N_ENT), lambda i, *_: (i, 0, 0)),
        scratch_shapes=[pltpu.VMEM((P_SENT, D_DIM), jnp.float32)],
    )
    outT = pl.pallas_call(
        _body,
        grid_spec=grid_spec,
        out_shape=jax.ShapeDtypeStruct((BATCH, D_DIM, N_ENT), jnp.float32),
        compiler_params=pltpu.CompilerParams(
            dimension_semantics=("arbitrary",)),
    )(starts, counts, perm, encoded_sents, U.T, V.T, W, ht, kt)

    return jnp.transpose(outT, (0, 2, 1))
